# baseline (speedup 1.0000x reference)
import functools

import jax
import jax.numpy as jnp
from jax import lax
from jax.experimental import pallas as pl
from jax.experimental.pallas import tpu as pltpu
from jax.experimental.pallas import tpu_sc as plsc

B = 4096
L = 26
D = 64
V = 100000

NC = 2
NS = 16
NW = NC * NS
CPW = D // NW
HB = B // 2
NT = 2 * L
NBI = 11
NBO = 4


def _sc_body(xT_hbm, tT_hbm, outT_hbm, row_v,
             i0, i1, i2, i3, i4, i5, i6, i7, i8, i9, i10,
             o0, o1, o2, o3,
             si0, si1, si2, si3, si4, si5, si6, si7, si8, si9, si10,
             so0, so1, so2, so3):
    wid = lax.axis_index("s") * NC + lax.axis_index("c")
    idx_v = (i0, i1, i2, i3, i4, i5, i6, i7, i8, i9, i10)
    outb_v = (o0, o1, o2, o3)
    sem_i = (si0, si1, si2, si3, si4, si5, si6, si7, si8, si9, si10)
    sem_o = (so0, so1, so2, so3)

    for ci in range(CPW):
        c = wid * CPW + ci
        idx_cp = [None] * NT
        out_cp = [None] * NT
        for p in range(NBI):
            idx_cp[p] = pltpu.async_copy(
                xT_hbm.at[p // 2, pl.ds((p % 2) * HB, HB)], idx_v[p],
                sem_i[p])
        pltpu.sync_copy(tT_hbm.at[c, pl.ds(0, V)], row_v)
        for t in range(NT):
            l, h = t // 2, t % 2
            pi = t % NBI
            po = t % NBO
            idx_cp[t].wait()
            if t >= NBO:
                out_cp[t - NBO].wait()

            @plsc.parallel_loop(0, HB, step=16, unroll=8)
            def _gather(off):
                idx = idx_v[pi][pl.ds(off, 16)]
                outb_v[po][pl.ds(off, 16)] = plsc.load_gather(row_v, [idx])

            out_cp[t] = pltpu.async_copy(
                outb_v[po], outT_hbm.at[l, c, pl.ds(h * HB, HB)], sem_o[po])
            if t + NBI < NT:
                tn = t + NBI
                idx_cp[tn] = pltpu.async_copy(
                    xT_hbm.at[tn // 2, pl.ds((tn % 2) * HB, HB)], idx_v[pi],
                    sem_i[pi])
        for t in range(NBO):
            out_cp[NT - NBO + t].wait()


@jax.jit
def kernel(x, table):
    xT = x.T.astype(jnp.int32)
    tT = table.T
    mesh = plsc.VectorSubcoreMesh(core_axis_name="c", subcore_axis_name="s")
    outT = pl.kernel(
        _sc_body,
        out_type=jax.ShapeDtypeStruct((L, D, B), jnp.float32),
        mesh=mesh,
        compiler_params=pltpu.CompilerParams(
            needs_layout_passes=False, use_tc_tiling_on_sc=True
        ),
        scratch_types=(
            [pltpu.VMEM((V,), jnp.float32)]
            + [pltpu.VMEM((HB,), jnp.int32)] * NBI
            + [pltpu.VMEM((HB,), jnp.float32)] * NBO
            + [pltpu.SemaphoreType.DMA] * (NBI + NBO)
        ),
    )(xT, tT)
    return jnp.transpose(outT, (2, 1, 0))


# speedup vs baseline: 1.0669x; 1.0669x over previous
import functools

import jax
import jax.numpy as jnp
from jax import lax
from jax.experimental import pallas as pl
from jax.experimental.pallas import tpu as pltpu
from jax.experimental.pallas import tpu_sc as plsc

B = 4096
L = 26
D = 64
V = 100000

NC = 2
NS = 16
NW = NC * NS
CPW = D // NW
NJ = B // 16
NBI = 5
NBO = 2


def _sc_body(xT_hbm, tT_hbm, outT_hbm, row_v, idx0_v, idx1_v, idx2_v,
             idx3_v, idx4_v, outb0_v, outb1_v,
             sem_i0, sem_i1, sem_i2, sem_i3, sem_i4, sem_o0, sem_o1):
    wid = lax.axis_index("s") * NC + lax.axis_index("c")
    idx_v = (idx0_v, idx1_v, idx2_v, idx3_v, idx4_v)
    outb_v = (outb0_v, outb1_v)
    sem_i = (sem_i0, sem_i1, sem_i2, sem_i3, sem_i4)
    sem_o = (sem_o0, sem_o1)

    for ci in range(CPW):
        c = wid * CPW + ci
        idx_cp = [None] * L
        out_cp = [None] * L
        for p in range(NBI):
            idx_cp[p] = pltpu.async_copy(
                xT_hbm.at[p, pl.ds(0, B)], idx_v[p], sem_i[p])
        pltpu.sync_copy(tT_hbm.at[c, pl.ds(0, V)], row_v)
        for l in range(L):
            pi = l % NBI
            po = l % NBO
            idx_cp[l].wait()
            if l >= NBO:
                out_cp[l - NBO].wait()

            @plsc.parallel_loop(0, B, step=16, unroll=8)
            def _gather(off):
                idx = idx_v[pi][pl.ds(off, 16)]
                outb_v[po][pl.ds(off, 16)] = plsc.load_gather(row_v, [idx])

            out_cp[l] = pltpu.async_copy(
                outb_v[po], outT_hbm.at[l, c, pl.ds(0, B)], sem_o[po])
            if l + NBI < L:
                idx_cp[l + NBI] = pltpu.async_copy(
                    xT_hbm.at[l + NBI, pl.ds(0, B)], idx_v[pi],
                    sem_i[pi])
        for t in range(NBO):
            out_cp[L - NBO + t].wait()


@jax.jit
def kernel(x, table):
    xT = x.T.astype(jnp.int32)
    tT = table.T
    mesh = plsc.VectorSubcoreMesh(core_axis_name="c", subcore_axis_name="s")
    outT = pl.kernel(
        _sc_body,
        out_type=jax.ShapeDtypeStruct((L, D, B), jnp.float32),
        mesh=mesh,
        compiler_params=pltpu.CompilerParams(
            needs_layout_passes=False, use_tc_tiling_on_sc=True
        ),
        scratch_types=[
            pltpu.VMEM((V,), jnp.float32),
            pltpu.VMEM((B,), jnp.int32),
            pltpu.VMEM((B,), jnp.int32),
            pltpu.VMEM((B,), jnp.int32),
            pltpu.VMEM((B,), jnp.int32),
            pltpu.VMEM((B,), jnp.int32),
            pltpu.VMEM((B,), jnp.float32),
            pltpu.VMEM((B,), jnp.float32),
            pltpu.SemaphoreType.DMA,
            pltpu.SemaphoreType.DMA,
            pltpu.SemaphoreType.DMA,
            pltpu.SemaphoreType.DMA,
            pltpu.SemaphoreType.DMA,
            pltpu.SemaphoreType.DMA,
            pltpu.SemaphoreType.DMA,
        ],
    )(xT, tT)
    return jnp.transpose(outT, (2, 1, 0))
